# Initial kernel scaffold; baseline (speedup 1.0000x reference)
import jax
import jax.numpy as jnp
from jax.experimental import pallas as pl


def kernel(input_imgs, token_type_ids, W_img, b_img, word_emb, pos_emb, tok_type_emb, ln_g, ln_b):
    raise NotImplementedError("write your pallas kernel here")


# speedup vs baseline: 1.3266x; 1.3266x over previous
import functools

import jax
import jax.numpy as jnp
from jax.experimental import pallas as pl

NUM_IMAGE_EMBEDS = 3
IMG_HIDDEN = 2048
HIDDEN = 768
CLS_ID = 101
SEP_ID = 102
LN_EPS = 1e-12
SEQ = NUM_IMAGE_EMBEDS + 2

BLOCK_B = 256


def _ln(x, g, b):
    mu = jnp.mean(x, axis=-1, keepdims=True)
    xc = x - mu
    var = jnp.mean(xc * xc, axis=-1, keepdims=True)
    return xc * jax.lax.rsqrt(var + LN_EPS) * g + b


def _body(x_ref, tt_ref, w_ref, bimg_ref, pos_ref, tte_ref, cls_ref, sep_ref,
          g_ref, b_ref, out_ref):
    bb = x_ref.shape[0]
    x = x_ref[...].reshape(bb * NUM_IMAGE_EMBEDS, IMG_HIDDEN)
    w = w_ref[...]
    proj = jnp.dot(x, w, preferred_element_type=jnp.float32)
    proj = proj.reshape(bb, NUM_IMAGE_EMBEDS, HIDDEN)

    g = g_ref[...]
    b = b_ref[...]
    tte = tte_ref[...]
    pos = pos_ref[...]
    tt = tt_ref[...]
    ttf = tt.astype(jnp.float32)
    dtte = (tte[1] - tte[0])[None, None, :]

    mid = (proj + bimg_ref[...][None, :, :]
           + pos[None, 1:1 + NUM_IMAGE_EMBEDS, :]
           + tte[0][None, None, :]
           + ttf[:, 1:1 + NUM_IMAGE_EMBEDS, None] * dtte)
    mid = _ln(mid, g[None], b[None])

    cands = jnp.concatenate([
        cls_ref[...] + pos[0:1, :] + tte[0:1, :],
        cls_ref[...] + pos[0:1, :] + tte[1:2, :],
        sep_ref[...] + pos[SEQ - 1:SEQ, :] + tte[0:1, :],
        sep_ref[...] + pos[SEQ - 1:SEQ, :] + tte[1:2, :],
    ], axis=0)
    cands = _ln(cands, g, b)
    row0 = jnp.where(tt[:, 0:1] == 0, cands[0:1, :], cands[1:2, :])
    row4 = jnp.where(tt[:, SEQ - 1:SEQ] == 0, cands[2:3, :], cands[3:4, :])

    out_ref[:, 1:1 + NUM_IMAGE_EMBEDS, :] = mid
    out_ref[:, 0:1, :] = row0[:, None, :]
    out_ref[:, SEQ - 1:SEQ, :] = row4[:, None, :]


@functools.partial(jax.jit, static_argnames=())
def kernel(input_imgs, token_type_ids, W_img, b_img, word_emb, pos_emb,
           tok_type_emb, ln_g, ln_b):
    bsz = input_imgs.shape[0]
    tt = token_type_ids.astype(jnp.int32)
    pos5 = pos_emb[:SEQ]
    cls_row = word_emb[CLS_ID][None, :]
    sep_row = word_emb[SEP_ID][None, :]
    bimg = b_img[None, :]
    g = ln_g[None, :]
    b = ln_b[None, :]

    grid = (bsz // BLOCK_B,)
    out = pl.pallas_call(
        _body,
        grid=grid,
        in_specs=[
            pl.BlockSpec((BLOCK_B, NUM_IMAGE_EMBEDS, IMG_HIDDEN),
                         lambda i: (i, 0, 0)),
            pl.BlockSpec((BLOCK_B, SEQ), lambda i: (i, 0)),
            pl.BlockSpec((IMG_HIDDEN, HIDDEN), lambda i: (0, 0)),
            pl.BlockSpec((1, HIDDEN), lambda i: (0, 0)),
            pl.BlockSpec((SEQ, HIDDEN), lambda i: (0, 0)),
            pl.BlockSpec((2, HIDDEN), lambda i: (0, 0)),
            pl.BlockSpec((1, HIDDEN), lambda i: (0, 0)),
            pl.BlockSpec((1, HIDDEN), lambda i: (0, 0)),
            pl.BlockSpec((1, HIDDEN), lambda i: (0, 0)),
            pl.BlockSpec((1, HIDDEN), lambda i: (0, 0)),
        ],
        out_specs=pl.BlockSpec((BLOCK_B, SEQ, HIDDEN), lambda i: (i, 0, 0)),
        out_shape=jax.ShapeDtypeStruct((bsz, SEQ, HIDDEN), jnp.float32),
    )(input_imgs, tt, W_img, bimg, pos5, tok_type_emb, cls_row, sep_row, g, b)
    return out
